# baseline (speedup 1.0000x reference)
import math

import jax
import jax.numpy as jnp
from jax.experimental import pallas as pl
from jax.experimental.pallas import tpu as pltpu
from jax.experimental.pallas import tpu_sc as plsc

T = 2048
D = 1024
H, C = 16, 64
QCD = 256
TOPK = 32
NI, DI = 4, 64
LMAX = 16
P = 512
NWIN = 128
HS = 8

TB = 256
NTB = T // TB
PB = 512
NPB = P // PB
NEG = -1e30

f32 = jnp.float32
bf16 = jnp.bfloat16
i32 = jnp.int32


def _dot(a, b):
    return jax.lax.dot_general(a, b, (((1,), (0,)), ((), ())),
                               preferred_element_type=f32)


def _rope(x, cos, sin):
    half = x.shape[-1] // 2
    rot = jnp.concatenate([-x[:, half:], x[:, :half]], axis=1)
    return x * cos + rot * sin


def _rope_wide(x, cos_t, sin_t, nh):
    parts = []
    for hh in range(nh):
        parts.append(-x[:, 64 * hh + 32:64 * hh + 64])
        parts.append(x[:, 64 * hh:64 * hh + 32])
    rot = jnp.concatenate(parts, axis=1)
    return x * cos_t + rot * sin_t


def _zc4_body(h_ref, nw_ref, wzc_ref, zc4_ref):
    hb = h_ref[...]
    ms = jnp.mean(hb * hb, axis=1, keepdims=True)
    x = hb * jax.lax.rsqrt(ms + 1e-6) * nw_ref[...]
    zc4_ref[...] = _dot(x.astype(bf16), wzc_ref[...])


def _proj_body(h_ref, nw_ref, qnwt_ref, cs_ref, bd_ref, wdq_ref,
               wiuq_ref, wuq_ref, www_ref, wsq_ref, wsk_ref, wsv_ref,
               qi_ref, wh_ref, qrope_ref, xqs_ref, xksT_ref,
               xvs_ref):
    hb = h_ref[...]
    ms = jnp.mean(hb * hb, axis=1, keepdims=True)
    x = hb * jax.lax.rsqrt(ms + 1e-6) * nw_ref[...]
    xb = x.astype(bf16)
    csc = cs_ref[...]
    cos, sin = csc[:, :64], csc[:, 64:]
    cosq = jnp.tile(cos, (1, H))
    sinq = jnp.tile(sin, (1, H))
    coss = jnp.tile(cos, (1, HS))
    sins = jnp.tile(sin, (1, HS))
    qlat = _dot(xb, wdq_ref[...])
    qlb = qlat.astype(bf16)
    qi_ref[...] = _dot(qlb, wiuq_ref[...]).astype(bf16)
    wh_ref[...] = _dot(xb, www_ref[...])
    qraw = _dot(qlb, wuq_ref[...])
    sq = (qraw * qraw).astype(bf16)
    qn = qraw * jax.lax.rsqrt(_dot(sq, bd_ref[...]) + 1e-6) * qnwt_ref[...]
    qrope_ref[...] = _rope_wide(qn, cosq, sinq, H).astype(bf16)
    xqs = _dot(xb, wsq_ref[...])
    xks = _dot(xb, wsk_ref[...])
    xqs_ref[...] = (_rope_wide(xqs, coss, sins, HS)
                    * (1.0 / math.sqrt(C))).astype(bf16)
    xksT_ref[...] = jnp.transpose(_rope_wide(xks, coss, sins, HS)).astype(bf16)
    xvs_ref[...] = _dot(xb, wsv_ref[...]).astype(bf16)


GW = 128
GW2 = 128


def _sc_gather(zc4, idx_row, cs, endp_row):
    mesh = plsc.VectorSubcoreMesh(core_axis_name="core",
                                  subcore_axis_name="subcore")

    def body(zc4_hbm, idx_hbm, cs_hbm, endp_hbm, gth_hbm, cse_hbm):
        def gat(i_vmem, o_vmem):
            pltpu.sync_copy(zc4_hbm.at[i_vmem.at[0]], o_vmem)

        pltpu.emit_pipeline(
            gat,
            grid=(P * LMAX // GW,),
            in_specs=[pl.BlockSpec((1, GW), lambda i: (0, i))],
            out_specs=[pl.BlockSpec((GW, 256), lambda i: (i, 0))],
            core_axis_name=("core", "subcore"),
            dimension_semantics=(pltpu.PARALLEL,),
        )(idx_hbm, gth_hbm)

        def gat2(i_vmem, o_vmem):
            pltpu.sync_copy(cs_hbm.at[i_vmem.at[0]], o_vmem)

        pltpu.emit_pipeline(
            gat2,
            grid=(P // GW2,),
            in_specs=[pl.BlockSpec((1, GW2), lambda i: (0, i))],
            out_specs=[pl.BlockSpec((GW2, 128), lambda i: (i, 0))],
            core_axis_name="subcore",
            dimension_semantics=(pltpu.PARALLEL,),
        )(endp_hbm, cse_hbm)

    return pl.kernel(
        body,
        out_type=[
            jax.ShapeDtypeStruct((P * LMAX, 256), f32),
            jax.ShapeDtypeStruct((P, 128), f32),
        ],
        mesh=mesh,
    )(zc4, idx_row, cs, endp_row)


def _compress_body(gth_ref, cse_ref, bkv_ref, bik_ref, knw_ref,
                   ccomp_ref, kidxT_ref, kallT_ref):
    g = gth_ref[...]

    def slot(l, lo):
        return g[l * PB:(l + 1) * PB, lo:lo + 64]

    mkv = jnp.full((PB, 64), NEG, f32)
    mik = jnp.full((PB, 64), NEG, f32)
    for l in range(LMAX):
        mkv = jnp.maximum(mkv, slot(l, 64) + bkv_ref[l:l + 1, :])
        mik = jnp.maximum(mik, slot(l, 192) + bik_ref[l:l + 1, :])
    skv = jnp.zeros((PB, 64), f32)
    sik = jnp.zeros((PB, 64), f32)
    akv = jnp.zeros((PB, 64), f32)
    aik = jnp.zeros((PB, 64), f32)
    for l in range(LMAX):
        ekv = jnp.exp(slot(l, 64) + bkv_ref[l:l + 1, :] - mkv)
        eik = jnp.exp(slot(l, 192) + bik_ref[l:l + 1, :] - mik)
        skv += ekv
        sik += eik
        akv += ekv * slot(l, 0)
        aik += eik * slot(l, 128)
    ccomp = akv / skv
    ccomp_ref[...] = ccomp.astype(bf16)
    kidxT_ref[...] = jnp.transpose(aik / sik).astype(bf16)

    cse = cse_ref[...]
    mean = jnp.mean(ccomp * ccomp, axis=1, keepdims=True)
    kn = ccomp * jax.lax.rsqrt(mean + 1e-6) * knw_ref[...]
    kall = _rope(kn, cse[:, :64], cse[:, 64:]) * (1.0 / math.sqrt(C))
    kallT_ref[...] = jnp.transpose(kall).astype(bf16)


def _swin_body(xqs_ref, xksT_ref, xksTp_ref, xvs_ref, xvsp_ref, wo_ref,
               osw_ref):
    blk = pl.program_id(0)
    isub = jax.lax.broadcasted_iota(i32, (TB, TB), 0)
    jlan = jax.lax.broadcasted_iota(i32, (TB, TB), 1)
    mask_cur = (jlan <= isub) & (isub - jlan < NWIN)
    mask_prev = (isub < jlan - (TB - NWIN)) & (blk > 0)
    ones64 = jnp.ones((TB, 64), bf16)
    outs = []
    for hh in range(HS):
        sl = slice(64 * hh, 64 * hh + 64)
        q = xqs_ref[:, sl]
        lc = _dot(q, xksT_ref[sl, :])
        lp = _dot(q, xksTp_ref[sl, :])
        lc = jnp.where(mask_cur, lc, NEG)
        lp = jnp.where(mask_prev, lp, NEG)
        m = jnp.maximum(jnp.max(lc, axis=1, keepdims=True),
                        jnp.max(lp, axis=1, keepdims=True))
        ec = jnp.exp(lc - m).astype(bf16)
        ep = jnp.exp(lp - m).astype(bf16)
        vac = jnp.concatenate([xvs_ref[:, sl], ones64], axis=1)
        vap = jnp.concatenate([xvsp_ref[:, sl], ones64], axis=1)
        oa = _dot(ec, vac) + _dot(ep, vap)
        outs.append(oa[:, :64] * (1.0 / oa[:, 64:65]))
    ocat = jnp.concatenate(outs, axis=1).astype(bf16)
    osw_ref[...] = _dot(ocat, wo_ref[...])


def _sel_body(qi_ref, wh_ref, kidxT_ref, endp_ref, qrope_ref, kallT_ref,
              ccomp_ref, sink_ref, wo_ref, h_ref, osw_ref, out_ref):
    kiT = kidxT_ref[...]
    scores = jnp.zeros((TB, P), f32)
    for ih in range(NI):
        s = _dot(qi_ref[:, 64 * ih:64 * ih + 64], kiT)
        scores += jnp.maximum(s, 0.0) * wh_ref[:, ih:ih + 1]
    t0 = pl.program_id(0) * TB
    rowpos = (t0 + jax.lax.broadcasted_iota(i32, (TB, 1), 0)).astype(f32)
    vis = endp_ref[...] < rowpos
    scores = jnp.where(vis, scores, NEG)
    sb = jax.lax.bitcast_convert_type(scores, i32)
    int_min = jnp.int32(-2147483648)
    uk = jnp.where(sb >= 0, sb, int_min - sb)
    lo = jnp.full((TB, 1), int_min, i32)
    hi = jnp.full((TB, 1), 2147483647, i32)
    for _ in range(32):
        mid = (lo >> 1) + (hi >> 1) + (lo & hi & 1)
        cnt = jnp.sum(jnp.where(uk >= mid, 1.0, 0.0), axis=1, keepdims=True)
        ge = cnt >= TOPK
        lo = jnp.where(ge, mid, lo)
        hi = jnp.where(ge, hi, mid)
    gt = uk > lo
    eq = uk == lo
    cnt_gt = jnp.sum(jnp.where(gt, 1.0, 0.0), axis=1, keepdims=True)
    li = jax.lax.broadcasted_iota(i32, (P, P), 0)
    lj = jax.lax.broadcasted_iota(i32, (P, P), 1)
    ltri = (li < lj).astype(bf16)
    prefix = _dot(eq.astype(bf16), ltri)
    selb = (gt | (eq & (prefix < (TOPK - cnt_gt)))) & (scores > -1e29)
    ma = jnp.where(selb, 0.0, NEG)

    kT = kallT_ref[...]
    vb = ccomp_ref[...]
    outs = []
    for hh in range(H):
        q = qrope_ref[:, 64 * hh:64 * hh + 64]
        e = jnp.exp(_dot(q, kT) + ma)
        sk = sink_ref[0:1, hh:hh + 1]
        inv = 1.0 / (jnp.sum(e, axis=1, keepdims=True) + jnp.exp(sk))
        outs.append(_dot(e.astype(bf16), vb) * inv)
    att = jnp.concatenate(outs, axis=1).astype(bf16)
    out_ref[...] = _dot(att, wo_ref[...]) + h_ref[...] + osw_ref[...]


def _full(shape):
    return pl.BlockSpec(shape, lambda i: (0, 0))


def _blk(shape):
    return pl.BlockSpec(shape, lambda i: (i, 0))


def _blkT(shape):
    return pl.BlockSpec(shape, lambda i: (0, i))


def _prevT(shape):
    return pl.BlockSpec(shape, lambda i: (0, jnp.maximum(i - 1, 0)))


def _prev(shape):
    return pl.BlockSpec(shape, lambda i: (jnp.maximum(i - 1, 0), 0))


def kernel(h, phrase_mask, phrase_token_idx, phrase_end_pos, rope_cos,
           rope_sin, W_dq, W_uq, kv_Wkv, kv_Wz, kv_Bpos, ik_Wkv, ik_Wz,
           ik_Bpos, idx_Wiuq, idx_Ww, qn_w, kn_w, W_o, sink_logits, norm_w,
           sw_Wq, sw_Wk, sw_Wv, sw_Wo):
    h2 = h[0]
    wzc = jnp.concatenate([kv_Wkv, kv_Wz, ik_Wkv, ik_Wz], axis=1).astype(bf16)
    www = jnp.pad(idx_Ww, ((0, 0), (0, 128 - NI))).astype(bf16)
    tok3 = phrase_token_idx[0].astype(i32).reshape(NPB, PB, LMAX)
    idxp = tok3.transpose(0, 2, 1).reshape(P * LMAX, 1)
    endp_i = phrase_end_pos[0].astype(i32).reshape(P, 1)
    endp_f = phrase_end_pos[0].astype(f32).reshape(1, P)
    cs = jnp.concatenate([rope_cos, rope_sin], axis=1)
    nw = norm_w.reshape(1, D)
    qnw = qn_w.reshape(1, C)
    knw = kn_w.reshape(1, C)
    sink = sink_logits.reshape(1, H)

    qnwt = jnp.tile(qn_w, H).reshape(1, D)
    eye16 = jnp.eye(H, dtype=f32)
    bd = (jnp.kron(eye16, jnp.ones((C, C), f32)) / C).astype(bf16)

    zc4 = pl.pallas_call(
        _zc4_body,
        grid=(NTB,),
        in_specs=[_blk((TB, D)), _full((1, D)), _full((D, 256))],
        out_specs=_blk((TB, 256)),
        out_shape=jax.ShapeDtypeStruct((T, 256), f32),
    )(h2, nw, wzc)

    gth, cseg = _sc_gather(zc4, idxp.reshape(1, P * LMAX), cs,
                           endp_i.reshape(1, P))

    qi, wh, qrope, xqs, xksT, xvs = pl.pallas_call(
        _proj_body,
        grid=(NTB,),
        in_specs=[
            _blk((TB, D)), _full((1, D)), _full((1, D)), _blk((TB, 128)),
            _full((D, D)),
            _full((D, QCD)), _full((QCD, 256)),
            _full((QCD, D)), _full((D, 128)), _full((D, 512)),
            _full((D, 512)), _full((D, 512)),
        ],
        out_specs=[
            _blk((TB, 256)), _blk((TB, 128)),
            _blk((TB, D)), _blk((TB, 512)), _blkT((512, TB)),
            _blk((TB, 512)),
        ],
        out_shape=[
            jax.ShapeDtypeStruct((T, 256), bf16),
            jax.ShapeDtypeStruct((T, 128), f32),
            jax.ShapeDtypeStruct((T, D), bf16),
            jax.ShapeDtypeStruct((T, 512), bf16),
            jax.ShapeDtypeStruct((512, T), bf16),
            jax.ShapeDtypeStruct((T, 512), bf16),
        ],
    )(h2, nw, qnwt, cs, bd, W_dq.astype(bf16), idx_Wiuq.astype(bf16),
      W_uq.astype(bf16), www, sw_Wq.astype(bf16), sw_Wk.astype(bf16),
      sw_Wv.astype(bf16))

    ccomp, kidxT, kallT = pl.pallas_call(
        _compress_body,
        grid=(NPB,),
        in_specs=[
            _blk((PB * LMAX, 256)), _blk((PB, 128)),
            _full((LMAX, C)), _full((LMAX, C)), _full((1, C)),
        ],
        out_specs=[_blk((PB, C)), _blkT((C, PB)), _blkT((C, PB))],
        out_shape=[
            jax.ShapeDtypeStruct((P, C), bf16),
            jax.ShapeDtypeStruct((C, P), bf16),
            jax.ShapeDtypeStruct((C, P), bf16),
        ],
    )(gth, cseg, kv_Bpos, ik_Bpos, knw)

    osw = pl.pallas_call(
        _swin_body,
        grid=(NTB,),
        in_specs=[
            _blk((TB, 512)), _blkT((512, TB)), _prevT((512, TB)),
            _blk((TB, 512)), _prev((TB, 512)), _full((512, D)),
        ],
        out_specs=_blk((TB, D)),
        out_shape=jax.ShapeDtypeStruct((T, D), f32),
    )(xqs, xksT, xksT, xvs, xvs, sw_Wo.astype(bf16))

    out = pl.pallas_call(
        _sel_body,
        grid=(NTB,),
        in_specs=[
            _blk((TB, 256)), _blk((TB, 128)), _full((C, P)), _full((1, P)),
            _blk((TB, D)), _full((C, P)), _full((P, C)), _full((1, H)),
            _full((D, D)), _blk((TB, D)), _blk((TB, D)),
        ],
        out_specs=_blk((TB, D)),
        out_shape=jax.ShapeDtypeStruct((T, D), f32),
    )(qi, wh, kidxT, endp_f, qrope, kallT, ccomp, sink, W_o.astype(bf16),
      h2, osw)

    return out.reshape(1, T, D)


# speedup vs baseline: 1.0160x; 1.0160x over previous
import math

import jax
import jax.numpy as jnp
from jax.experimental import pallas as pl
from jax.experimental.pallas import tpu as pltpu
from jax.experimental.pallas import tpu_sc as plsc

T = 2048
D = 1024
H, C = 16, 64
QCD = 256
TOPK = 32
NI, DI = 4, 64
LMAX = 16
P = 512
NWIN = 128
HS = 8

TB = 256
NTB = T // TB
PB = 512
NPB = P // PB
NEG = -1e30

f32 = jnp.float32
bf16 = jnp.bfloat16
i32 = jnp.int32


def _dot(a, b):
    return jax.lax.dot_general(a, b, (((1,), (0,)), ((), ())),
                               preferred_element_type=f32)


def _rope(x, cos, sin):
    half = x.shape[-1] // 2
    rot = jnp.concatenate([-x[:, half:], x[:, :half]], axis=1)
    return x * cos + rot * sin


def _rope_wide(x, cos_t, sin_t, nh):
    parts = []
    for hh in range(nh):
        parts.append(-x[:, 64 * hh + 32:64 * hh + 64])
        parts.append(x[:, 64 * hh:64 * hh + 32])
    rot = jnp.concatenate(parts, axis=1)
    return x * cos_t + rot * sin_t


def _zc4_body(h_ref, nw_ref, wzc_ref, zc4_ref):
    hb = h_ref[...]
    ms = jnp.mean(hb * hb, axis=1, keepdims=True)
    x = hb * jax.lax.rsqrt(ms + 1e-6) * nw_ref[...]
    zc4_ref[...] = _dot(x.astype(bf16), wzc_ref[...])


def _proj_body(h_ref, nw_ref, qnwt_ref, cs_ref, bd_ref, wdq_ref,
               wiuq_ref, wuq_ref, www_ref, wsq_ref, wsk_ref, wsv_ref,
               qi_ref, wh_ref, qrope_ref, xqs_ref, xksT_ref,
               xvs_ref):
    hb = h_ref[...]
    ms = jnp.mean(hb * hb, axis=1, keepdims=True)
    x = hb * jax.lax.rsqrt(ms + 1e-6) * nw_ref[...]
    xb = x.astype(bf16)
    csc = cs_ref[...]
    cos, sin = csc[:, :64], csc[:, 64:]
    cosq = jnp.tile(cos, (1, H))
    sinq = jnp.tile(sin, (1, H))
    coss = jnp.tile(cos, (1, HS))
    sins = jnp.tile(sin, (1, HS))
    qlat = _dot(xb, wdq_ref[...])
    qlb = qlat.astype(bf16)
    qi_ref[...] = _dot(qlb, wiuq_ref[...]).astype(bf16)
    wh_ref[...] = _dot(xb, www_ref[...])
    qraw = _dot(qlb, wuq_ref[...])
    sq = (qraw * qraw).astype(bf16)
    qn = qraw * jax.lax.rsqrt(_dot(sq, bd_ref[...]) + 1e-6) * qnwt_ref[...]
    qrope_ref[...] = _rope_wide(qn, cosq, sinq, H).astype(bf16)
    xqs = _dot(xb, wsq_ref[...])
    xks = _dot(xb, wsk_ref[...])
    xqs_ref[...] = (_rope_wide(xqs, coss, sins, HS)
                    * (1.0 / math.sqrt(C))).astype(bf16)
    xksT_ref[...] = jnp.transpose(_rope_wide(xks, coss, sins, HS)).astype(bf16)
    xvs_ref[...] = _dot(xb, wsv_ref[...]).astype(bf16)


GW = 128
GW2 = 128


def _sc_gather(zc4, idx_row, cs, endp_row):
    mesh = plsc.VectorSubcoreMesh(core_axis_name="core",
                                  subcore_axis_name="subcore")

    def body(zc4_hbm, idx_hbm, cs_hbm, endp_hbm, gth_hbm, cse_hbm):
        def gat(i_vmem, o_vmem):
            pltpu.sync_copy(zc4_hbm.at[i_vmem.at[0]], o_vmem)

        pltpu.emit_pipeline(
            gat,
            grid=(P * LMAX // GW,),
            in_specs=[pl.BlockSpec((1, GW), lambda i: (0, i))],
            out_specs=[pl.BlockSpec((GW, 256), lambda i: (i, 0))],
            core_axis_name=("core", "subcore"),
            dimension_semantics=(pltpu.PARALLEL,),
        )(idx_hbm, gth_hbm)

        def gat2(i_vmem, o_vmem):
            pltpu.sync_copy(cs_hbm.at[i_vmem.at[0]], o_vmem)

        pltpu.emit_pipeline(
            gat2,
            grid=(P // GW2,),
            in_specs=[pl.BlockSpec((1, GW2), lambda i: (0, i))],
            out_specs=[pl.BlockSpec((GW2, 128), lambda i: (i, 0))],
            core_axis_name="subcore",
            dimension_semantics=(pltpu.PARALLEL,),
        )(endp_hbm, cse_hbm)

    return pl.kernel(
        body,
        out_type=[
            jax.ShapeDtypeStruct((P * LMAX, 256), f32),
            jax.ShapeDtypeStruct((P, 128), f32),
        ],
        mesh=mesh,
    )(zc4, idx_row, cs, endp_row)


def _compress_into(gth_ref, cse_ref, bkv_ref, bik_ref, knw_ref,
                   ccomp_ref, kidxT_ref, kallT_ref):
    g = gth_ref[...]

    def slot(l, lo):
        return g[l * PB:(l + 1) * PB, lo:lo + 64]

    mkv = jnp.full((PB, 64), NEG, f32)
    mik = jnp.full((PB, 64), NEG, f32)
    for l in range(LMAX):
        mkv = jnp.maximum(mkv, slot(l, 64) + bkv_ref[l:l + 1, :])
        mik = jnp.maximum(mik, slot(l, 192) + bik_ref[l:l + 1, :])
    skv = jnp.zeros((PB, 64), f32)
    sik = jnp.zeros((PB, 64), f32)
    akv = jnp.zeros((PB, 64), f32)
    aik = jnp.zeros((PB, 64), f32)
    for l in range(LMAX):
        ekv = jnp.exp(slot(l, 64) + bkv_ref[l:l + 1, :] - mkv)
        eik = jnp.exp(slot(l, 192) + bik_ref[l:l + 1, :] - mik)
        skv += ekv
        sik += eik
        akv += ekv * slot(l, 0)
        aik += eik * slot(l, 128)
    ccomp = akv / skv
    ccomp_ref[...] = ccomp.astype(bf16)
    kidxT_ref[...] = jnp.transpose(aik / sik).astype(bf16)

    cse = cse_ref[...]
    mean = jnp.mean(ccomp * ccomp, axis=1, keepdims=True)
    kn = ccomp * jax.lax.rsqrt(mean + 1e-6) * knw_ref[...]
    kall = _rope(kn, cse[:, :64], cse[:, 64:]) * (1.0 / math.sqrt(C))
    kallT_ref[...] = jnp.transpose(kall).astype(bf16)


def _sel_body(qi_ref, wh_ref, endp_ref, qrope_ref, sink_ref, wo_ref,
              h_ref, xqs_ref, xksT_ref, xksTp_ref, xvs_ref, xvsp_ref,
              swo_ref, gth_ref, cse_ref, bkv_ref, bik_ref, knw_ref,
              out_ref, ccomp_s, kidxT_s, kallT_s):
    @pl.when(pl.program_id(0) == 0)
    def _():
        _compress_into(gth_ref, cse_ref, bkv_ref, bik_ref, knw_ref,
                       ccomp_s, kidxT_s, kallT_s)

    blk = pl.program_id(0)
    isub = jax.lax.broadcasted_iota(i32, (TB, TB), 0)
    jlan = jax.lax.broadcasted_iota(i32, (TB, TB), 1)
    mask_cur = (jlan <= isub) & (isub - jlan < NWIN)
    mask_prev = (isub < jlan - (TB - NWIN)) & (blk > 0)
    ones64 = jnp.ones((TB, 64), bf16)
    souts = []
    for hh in range(HS):
        sl = slice(64 * hh, 64 * hh + 64)
        q = xqs_ref[:, sl]
        lc = _dot(q, xksT_ref[sl, :])
        lp = _dot(q, xksTp_ref[sl, :])
        lc = jnp.where(mask_cur, lc, NEG)
        lp = jnp.where(mask_prev, lp, NEG)
        m = jnp.maximum(jnp.max(lc, axis=1, keepdims=True),
                        jnp.max(lp, axis=1, keepdims=True))
        ec = jnp.exp(lc - m).astype(bf16)
        ep = jnp.exp(lp - m).astype(bf16)
        vac = jnp.concatenate([xvs_ref[:, sl], ones64], axis=1)
        vap = jnp.concatenate([xvsp_ref[:, sl], ones64], axis=1)
        oa = _dot(ec, vac) + _dot(ep, vap)
        souts.append(oa[:, :64] * (1.0 / oa[:, 64:65]))
    osw = _dot(jnp.concatenate(souts, axis=1).astype(bf16), swo_ref[...])

    kiT = kidxT_s[...]
    scores = jnp.zeros((TB, P), f32)
    for ih in range(NI):
        s = _dot(qi_ref[:, 64 * ih:64 * ih + 64], kiT)
        scores += jnp.maximum(s, 0.0) * wh_ref[:, ih:ih + 1]
    t0 = pl.program_id(0) * TB
    rowpos = (t0 + jax.lax.broadcasted_iota(i32, (TB, 1), 0)).astype(f32)
    vis = endp_ref[...] < rowpos
    scores = jnp.where(vis, scores, NEG)
    sb = jax.lax.bitcast_convert_type(scores, i32)
    int_min = jnp.int32(-2147483648)
    uk = jnp.where(sb >= 0, sb, int_min - sb)
    lo = jnp.full((TB, 1), int_min, i32)
    hi = jnp.full((TB, 1), 2147483647, i32)
    for _ in range(32):
        mid = (lo >> 1) + (hi >> 1) + (lo & hi & 1)
        cnt = jnp.sum(jnp.where(uk >= mid, 1.0, 0.0), axis=1, keepdims=True)
        ge = cnt >= TOPK
        lo = jnp.where(ge, mid, lo)
        hi = jnp.where(ge, hi, mid)
    gt = uk > lo
    eq = uk == lo
    cnt_gt = jnp.sum(jnp.where(gt, 1.0, 0.0), axis=1, keepdims=True)
    li = jax.lax.broadcasted_iota(i32, (P, P), 0)
    lj = jax.lax.broadcasted_iota(i32, (P, P), 1)
    ltri = (li < lj).astype(bf16)
    prefix = _dot(eq.astype(bf16), ltri)
    selb = (gt | (eq & (prefix < (TOPK - cnt_gt)))) & (scores > -1e29)
    ma = jnp.where(selb, 0.0, NEG)

    kT = kallT_s[...]
    vb = ccomp_s[...]
    outs = []
    for hh in range(H):
        q = qrope_ref[:, 64 * hh:64 * hh + 64]
        e = jnp.exp(_dot(q, kT) + ma)
        sk = sink_ref[0:1, hh:hh + 1]
        inv = 1.0 / (jnp.sum(e, axis=1, keepdims=True) + jnp.exp(sk))
        outs.append(_dot(e.astype(bf16), vb) * inv)
    att = jnp.concatenate(outs, axis=1).astype(bf16)
    out_ref[...] = _dot(att, wo_ref[...]) + h_ref[...] + osw


def _full(shape):
    return pl.BlockSpec(shape, lambda i: (0, 0))


def _blk(shape):
    return pl.BlockSpec(shape, lambda i: (i, 0))


def _blkT(shape):
    return pl.BlockSpec(shape, lambda i: (0, i))


def _prevT(shape):
    return pl.BlockSpec(shape, lambda i: (0, jnp.maximum(i - 1, 0)))


def _prev(shape):
    return pl.BlockSpec(shape, lambda i: (jnp.maximum(i - 1, 0), 0))


def kernel(h, phrase_mask, phrase_token_idx, phrase_end_pos, rope_cos,
           rope_sin, W_dq, W_uq, kv_Wkv, kv_Wz, kv_Bpos, ik_Wkv, ik_Wz,
           ik_Bpos, idx_Wiuq, idx_Ww, qn_w, kn_w, W_o, sink_logits, norm_w,
           sw_Wq, sw_Wk, sw_Wv, sw_Wo):
    h2 = h[0]
    wzc = jnp.concatenate([kv_Wkv, kv_Wz, ik_Wkv, ik_Wz], axis=1).astype(bf16)
    www = jnp.pad(idx_Ww, ((0, 0), (0, 128 - NI))).astype(bf16)
    tok3 = phrase_token_idx[0].astype(i32).reshape(NPB, PB, LMAX)
    idxp = tok3.transpose(0, 2, 1).reshape(P * LMAX, 1)
    endp_i = phrase_end_pos[0].astype(i32).reshape(P, 1)
    endp_f = phrase_end_pos[0].astype(f32).reshape(1, P)
    cs = jnp.concatenate([rope_cos, rope_sin], axis=1)
    nw = norm_w.reshape(1, D)
    qnw = qn_w.reshape(1, C)
    knw = kn_w.reshape(1, C)
    sink = sink_logits.reshape(1, H)

    qnwt = jnp.tile(qn_w, H).reshape(1, D)
    eye16 = jnp.eye(H, dtype=f32)
    bd = (jnp.kron(eye16, jnp.ones((C, C), f32)) / C).astype(bf16)

    zc4 = pl.pallas_call(
        _zc4_body,
        grid=(NTB,),
        in_specs=[_blk((TB, D)), _full((1, D)), _full((D, 256))],
        out_specs=_blk((TB, 256)),
        out_shape=jax.ShapeDtypeStruct((T, 256), f32),
    )(h2, nw, wzc)

    gth, cseg = _sc_gather(zc4, idxp.reshape(1, P * LMAX), cs,
                           endp_i.reshape(1, P))

    qi, wh, qrope, xqs, xksT, xvs = pl.pallas_call(
        _proj_body,
        grid=(NTB,),
        in_specs=[
            _blk((TB, D)), _full((1, D)), _full((1, D)), _blk((TB, 128)),
            _full((D, D)),
            _full((D, QCD)), _full((QCD, 256)),
            _full((QCD, D)), _full((D, 128)), _full((D, 512)),
            _full((D, 512)), _full((D, 512)),
        ],
        out_specs=[
            _blk((TB, 256)), _blk((TB, 128)),
            _blk((TB, D)), _blk((TB, 512)), _blkT((512, TB)),
            _blk((TB, 512)),
        ],
        out_shape=[
            jax.ShapeDtypeStruct((T, 256), bf16),
            jax.ShapeDtypeStruct((T, 128), f32),
            jax.ShapeDtypeStruct((T, D), bf16),
            jax.ShapeDtypeStruct((T, 512), bf16),
            jax.ShapeDtypeStruct((512, T), bf16),
            jax.ShapeDtypeStruct((T, 512), bf16),
        ],
    )(h2, nw, qnwt, cs, bd, W_dq.astype(bf16), idx_Wiuq.astype(bf16),
      W_uq.astype(bf16), www, sw_Wq.astype(bf16), sw_Wk.astype(bf16),
      sw_Wv.astype(bf16))

    out = pl.pallas_call(
        _sel_body,
        grid=(NTB,),
        in_specs=[
            _blk((TB, 256)), _blk((TB, 128)), _full((1, P)),
            _blk((TB, D)), _full((1, H)), _full((D, D)), _blk((TB, D)),
            _blk((TB, 512)), _blkT((512, TB)), _prevT((512, TB)),
            _blk((TB, 512)), _prev((TB, 512)), _full((512, D)),
            _full((P * LMAX, 256)), _full((P, 128)),
            _full((LMAX, C)), _full((LMAX, C)), _full((1, C)),
        ],
        out_specs=_blk((TB, D)),
        out_shape=jax.ShapeDtypeStruct((T, D), f32),
        scratch_shapes=[
            pltpu.VMEM((P, C), bf16),
            pltpu.VMEM((C, P), bf16),
            pltpu.VMEM((C, P), bf16),
        ],
    )(qi, wh, endp_f, qrope, sink, W_o.astype(bf16), h2,
      xqs, xksT, xksT, xvs, xvs, sw_Wo.astype(bf16),
      gth, cseg, kv_Bpos, ik_Bpos, knw)

    return out.reshape(1, T, D)
